# baseline (speedup 1.0000x reference)
import math

import jax
import jax.numpy as jnp
from jax.experimental import pallas as pl
from jax.experimental.pallas import tpu as pltpu

B, S, HS = 1, 2048, 2048
H, KVH, DH = 16, 4, 128
GROUPS = H // KVH
THETA = 10000.0

BSR = 512
BSQ = 256


def _rot(x):
    return jnp.concatenate([-x[:, DH // 2:], x[:, :DH // 2]], axis=1)


def _qkv_body(x_ref, w_ref, cos_ref, sin_ref, q_ref, k_ref, v_ref):
    qkv = jnp.dot(x_ref[...], w_ref[...], preferred_element_type=jnp.float32)
    cos = cos_ref[...]
    sin = sin_ref[...]
    for h in range(H):
        qh = qkv[:, h * DH:(h + 1) * DH]
        q_ref[h] = (qh * cos + _rot(qh) * sin).astype(jnp.bfloat16)
    off = H * DH
    for g in range(KVH):
        kg = qkv[:, off + g * DH: off + (g + 1) * DH]
        k_ref[g] = (kg * cos + _rot(kg) * sin).astype(jnp.bfloat16)
    off2 = (H + KVH) * DH
    for g in range(KVH):
        v_ref[g] = qkv[:, off2 + g * DH: off2 + (g + 1) * DH].astype(jnp.bfloat16)


BSC = BSQ
NQ = S // BSQ
_I_TAB = [i for i in range(NQ) for _ in range(i + 1)]
_J_TAB = [j for i in range(NQ) for j in range(i + 1)]
NT = len(_I_TAB)


def _attn_body(i_tab, j_tab, q_ref, k_ref, v_ref, o_ref, m_s, l_s, acc_s):
    t = pl.program_id(1)
    i = i_tab[t]
    j = j_tab[t]
    q = q_ref[0]
    k = k_ref[0, pl.ds(j * BSC, BSC), :]
    v = v_ref[0, pl.ds(j * BSC, BSC), :]
    s = jax.lax.dot_general(q, k, (((1,), (1,)), ((), ())),
                            preferred_element_type=jnp.float32)
    s = s * (1.0 / math.sqrt(DH))
    row = jax.lax.broadcasted_iota(jnp.int32, (BSQ, BSC), 0)
    col = jax.lax.broadcasted_iota(jnp.int32, (BSQ, BSC), 1)
    s = jnp.where((j < i) | (col <= row), s, -1e30)

    @pl.when(j == 0)
    def _():
        m_s[...] = jnp.full((BSQ, 128), -jnp.inf, jnp.float32)
        l_s[...] = jnp.zeros((BSQ, 128), jnp.float32)
        acc_s[...] = jnp.zeros((BSQ, DH), jnp.float32)

    m_prev = m_s[:, 0:1]
    l_prev = l_s[:, 0:1]
    m_new = jnp.maximum(m_prev, jnp.max(s, axis=1, keepdims=True))
    alpha = jnp.exp(m_prev - m_new)
    p = jnp.exp(s - m_new)
    l_new = l_prev * alpha + jnp.sum(p, axis=1, keepdims=True)
    pv = jnp.dot(p.astype(jnp.bfloat16), v, preferred_element_type=jnp.float32)
    acc_new = acc_s[...] * alpha + pv
    m_s[...] = jnp.broadcast_to(m_new, (BSQ, 128))
    l_s[...] = jnp.broadcast_to(l_new, (BSQ, 128))
    acc_s[...] = acc_new

    @pl.when(j == i)
    def _():
        o_ref[...] = (acc_new / l_new).astype(jnp.bfloat16)


def _proj_body(o_ref, w_ref, out_ref):
    out_ref[...] = jnp.dot(o_ref[...], w_ref[...],
                           preferred_element_type=jnp.float32)


def kernel(hidden_states, position_ids, Wq, Wk, Wv, Wo):
    x = hidden_states[0].astype(jnp.bfloat16)
    w_qkv = jnp.concatenate([Wq, Wk, Wv], axis=1).astype(jnp.bfloat16)
    pos = position_ids[0].astype(jnp.float32)
    inv_freq = 1.0 / (THETA ** (jnp.arange(0, DH, 2, dtype=jnp.float32) / DH))
    freqs = pos[:, None] * inv_freq[None, :]
    emb = jnp.concatenate([freqs, freqs], axis=1)
    cos = jnp.cos(emb)
    sin = jnp.sin(emb)

    q, k, v = pl.pallas_call(
        _qkv_body,
        grid=(S // BSR,),
        in_specs=[
            pl.BlockSpec((BSR, HS), lambda i: (i, 0)),
            pl.BlockSpec((HS, (H + 2 * KVH) * DH), lambda i: (0, 0)),
            pl.BlockSpec((BSR, DH), lambda i: (i, 0)),
            pl.BlockSpec((BSR, DH), lambda i: (i, 0)),
        ],
        out_specs=[
            pl.BlockSpec((H, BSR, DH), lambda i: (0, i, 0)),
            pl.BlockSpec((KVH, BSR, DH), lambda i: (0, i, 0)),
            pl.BlockSpec((KVH, BSR, DH), lambda i: (0, i, 0)),
        ],
        out_shape=[
            jax.ShapeDtypeStruct((H, S, DH), jnp.bfloat16),
            jax.ShapeDtypeStruct((KVH, S, DH), jnp.bfloat16),
            jax.ShapeDtypeStruct((KVH, S, DH), jnp.bfloat16),
        ],
    )(x, w_qkv, cos, sin)

    i_tab = jnp.asarray(_I_TAB, jnp.int32)
    j_tab = jnp.asarray(_J_TAB, jnp.int32)
    o = pl.pallas_call(
        _attn_body,
        grid_spec=pltpu.PrefetchScalarGridSpec(
            num_scalar_prefetch=2,
            grid=(H, NT),
            in_specs=[
                pl.BlockSpec((1, BSQ, DH), lambda h, t, it, jt: (h, it[t], 0)),
                pl.BlockSpec((1, S, DH), lambda h, t, it, jt: (h // GROUPS, 0, 0)),
                pl.BlockSpec((1, S, DH), lambda h, t, it, jt: (h // GROUPS, 0, 0)),
            ],
            out_specs=pl.BlockSpec((BSQ, DH), lambda h, t, it, jt: (it[t], h)),
            scratch_shapes=[
                pltpu.VMEM((BSQ, 128), jnp.float32),
                pltpu.VMEM((BSQ, 128), jnp.float32),
                pltpu.VMEM((BSQ, DH), jnp.float32),
            ],
        ),
        out_shape=jax.ShapeDtypeStruct((S, H * DH), jnp.bfloat16),
    )(i_tab, j_tab, q, k, v)

    out = pl.pallas_call(
        _proj_body,
        grid=(S // BSR,),
        in_specs=[
            pl.BlockSpec((BSR, H * DH), lambda i: (i, 0)),
            pl.BlockSpec((H * DH, HS), lambda i: (0, 0)),
        ],
        out_specs=pl.BlockSpec((BSR, HS), lambda i: (i, 0)),
        out_shape=jax.ShapeDtypeStruct((S, HS), jnp.float32),
    )(o, Wo.astype(jnp.bfloat16))
    return out[None]


# speedup vs baseline: 1.9658x; 1.9658x over previous
import math

import jax
import jax.numpy as jnp
from jax.experimental import pallas as pl
from jax.experimental.pallas import tpu as pltpu

B, S, HS = 1, 2048, 2048
H, KVH, DH = 16, 4, 128
GROUPS = H // KVH
THETA = 10000.0

BSR = 512
BSQ = 256


def _rot(x):
    return jnp.concatenate([-x[:, DH // 2:], x[:, :DH // 2]], axis=1)


def _qkv_body(x_ref, w_ref, cos_ref, sin_ref, q_ref, k_ref, v_ref):
    qkv = jnp.dot(x_ref[...], w_ref[...], preferred_element_type=jnp.float32)
    cos = cos_ref[...]
    sin = sin_ref[...]
    for h in range(H):
        qh = qkv[:, h * DH:(h + 1) * DH]
        q_ref[h] = (qh * cos + _rot(qh) * sin).astype(jnp.bfloat16)
    off = H * DH
    for g in range(KVH):
        kg = qkv[:, off + g * DH: off + (g + 1) * DH]
        k_ref[g] = (kg * cos + _rot(kg) * sin).astype(jnp.bfloat16)
    off2 = (H + KVH) * DH
    for g in range(KVH):
        v_ref[g] = qkv[:, off2 + g * DH: off2 + (g + 1) * DH].astype(jnp.bfloat16)


def _attn_body(q_ref, k_ref, v_ref, o_ref):
    i = pl.program_id(1)
    q = q_ref[0]
    k = k_ref[0]
    s = jax.lax.dot_general(q, k, (((1,), (1,)), ((), ())),
                            preferred_element_type=jnp.float32)
    s = s * (1.0 / math.sqrt(DH))
    row = i * BSQ + jax.lax.broadcasted_iota(jnp.int32, (BSQ, S), 0)
    col = jax.lax.broadcasted_iota(jnp.int32, (BSQ, S), 1)
    s = jnp.where(col <= row, s, -1e30)
    m = jnp.max(s, axis=1, keepdims=True)
    p = jnp.exp(s - m)
    l = jnp.sum(p, axis=1, keepdims=True)
    acc = jnp.dot(p.astype(jnp.bfloat16), v_ref[0],
                  preferred_element_type=jnp.float32)
    o_ref[...] = (acc / l).astype(jnp.bfloat16)


def _proj_body(o_ref, w_ref, out_ref):
    out_ref[...] = jnp.dot(o_ref[...], w_ref[...],
                           preferred_element_type=jnp.float32)


def kernel(hidden_states, position_ids, Wq, Wk, Wv, Wo):
    x = hidden_states[0].astype(jnp.bfloat16)
    w_qkv = jnp.concatenate([Wq, Wk, Wv], axis=1).astype(jnp.bfloat16)
    pos = position_ids[0].astype(jnp.float32)
    inv_freq = 1.0 / (THETA ** (jnp.arange(0, DH, 2, dtype=jnp.float32) / DH))
    freqs = pos[:, None] * inv_freq[None, :]
    emb = jnp.concatenate([freqs, freqs], axis=1)
    cos = jnp.cos(emb)
    sin = jnp.sin(emb)

    q, k, v = pl.pallas_call(
        _qkv_body,
        grid=(S // BSR,),
        in_specs=[
            pl.BlockSpec((BSR, HS), lambda i: (i, 0)),
            pl.BlockSpec((HS, (H + 2 * KVH) * DH), lambda i: (0, 0)),
            pl.BlockSpec((BSR, DH), lambda i: (i, 0)),
            pl.BlockSpec((BSR, DH), lambda i: (i, 0)),
        ],
        out_specs=[
            pl.BlockSpec((H, BSR, DH), lambda i: (0, i, 0)),
            pl.BlockSpec((KVH, BSR, DH), lambda i: (0, i, 0)),
            pl.BlockSpec((KVH, BSR, DH), lambda i: (0, i, 0)),
        ],
        out_shape=[
            jax.ShapeDtypeStruct((H, S, DH), jnp.bfloat16),
            jax.ShapeDtypeStruct((KVH, S, DH), jnp.bfloat16),
            jax.ShapeDtypeStruct((KVH, S, DH), jnp.bfloat16),
        ],
    )(x, w_qkv, cos, sin)

    o = pl.pallas_call(
        _attn_body,
        grid=(H, S // BSQ),
        in_specs=[
            pl.BlockSpec((1, BSQ, DH), lambda h, i: (h, i, 0)),
            pl.BlockSpec((1, S, DH), lambda h, i: (h // GROUPS, 0, 0)),
            pl.BlockSpec((1, S, DH), lambda h, i: (h // GROUPS, 0, 0)),
        ],
        out_specs=pl.BlockSpec((BSQ, DH), lambda h, i: (i, h)),
        out_shape=jax.ShapeDtypeStruct((S, H * DH), jnp.bfloat16),
    )(q, k, v)

    out = pl.pallas_call(
        _proj_body,
        grid=(S // BSR,),
        in_specs=[
            pl.BlockSpec((BSR, H * DH), lambda i: (i, 0)),
            pl.BlockSpec((H * DH, HS), lambda i: (0, 0)),
        ],
        out_specs=pl.BlockSpec((BSR, HS), lambda i: (i, 0)),
        out_shape=jax.ShapeDtypeStruct((S, HS), jnp.float32),
    )(o, Wo.astype(jnp.bfloat16))
    return out[None]
